# baseline (speedup 1.0000x reference)
import jax
import jax.numpy as jnp
from jax import lax
from jax.experimental import pallas as pl
from jax.experimental.pallas import tpu as pltpu
from jax.experimental.pallas import tpu_sc as plsc

_N = 10000
_E = 320000
_D = 128
_EPS = 1e-5

_NC = 2
_NS = 16
_CHUNK = 80
_NITER = 250
_EPT = _NITER * _CHUNK
_NP = 10240
_DH = _D // _NC
_RPT = _NP // _NS
_ZR = 128

_BR = 2000
_G = _N // _BR


def _sc_pass(with_deg: bool):
  mesh = plsc.VectorSubcoreMesh(core_axis_name="c", subcore_axis_name="s",
                                num_cores=_NC, num_subcores=_NS)
  out_type = [jax.ShapeDtypeStruct((_NC, _NP, _DH), jnp.float32)]
  scratch = [
      pltpu.VMEM((_NITER, _CHUNK), jnp.int32),
      pltpu.VMEM((_NITER, _CHUNK), jnp.int32),
      [pltpu.VMEM((_CHUNK, _DH), jnp.float32)] * 2,
      pltpu.VMEM((_ZR, _DH), jnp.float32),
      pltpu.VMEM_SHARED((_NP, _DH), jnp.float32),
      [pltpu.SemaphoreType.DMA] * 2,
  ]
  if with_deg:
    out_type.append(jax.ShapeDtypeStruct((_NC, _NP, _DH), jnp.float32))
    scratch += [
        pltpu.VMEM((_NP,), jnp.float32),
        pltpu.VMEM((_RPT,), jnp.float32),
        pltpu.VMEM((_RPT,), jnp.float32),
        pltpu.VMEM_SHARED((_NS, _NP), jnp.float32),
    ]

  def body(y_hbm, src_hbm, dst_hbm, out_hbm, *rest):
    if with_deg:
      (degb_hbm, src_v, dst_v, bufs, zbuf, acc, gsems,
       degp, degacc, tbuf, stage_sh) = rest
    else:
      src_v, dst_v, bufs, zbuf, acc, gsems = rest
    c = lax.axis_index("c")
    s = lax.axis_index("s")
    row0 = s * _RPT
    yv = y_hbm.at[c]
    zeros16 = jnp.zeros((16,), jnp.float32)
    ones16 = jnp.ones((16,), jnp.float32)

    def _zb(i, _):
      zbuf[i // (_DH // 16), pl.ds((i % (_DH // 16)) * 16, 16)] = zeros16
      return 0
    lax.fori_loop(0, _ZR * (_DH // 16), _zb, 0)
    for k in range(_RPT // _ZR):
      pltpu.sync_copy(zbuf, acc.at[pl.ds(row0 + k * _ZR, _ZR)])

    if with_deg:
      def _zd(i, _):
        degp[pl.ds(i * 16, 16)] = zeros16
        return 0
      lax.fori_loop(0, _NP // 16, _zd, 0)

    pltpu.sync_copy(src_hbm.at[s], src_v)
    pltpu.sync_copy(dst_hbm.at[s], dst_v)
    plsc.subcore_barrier()

    def _gather(i, b):
      pltpu.async_copy(yv.at[src_v.at[i]], bufs[b], gsems[b])

    def _wait_g(b):
      pltpu.make_async_copy(yv.at[pl.ds(0, _CHUNK)], bufs[b], gsems[b]).wait()

    def _work(i, b):
      _wait_g(b)
      pltpu.sync_copy(bufs[b], acc.at[dst_v.at[i]], add=True)
      if with_deg:
        @pl.when((i // (_NITER // 2)) == c)
        def _():
          for k in range(_CHUNK // 16):
            d16 = dst_v[i, pl.ds(k * 16, 16)]
            plsc.addupdate_scatter(degp, [d16], ones16)

    _gather(0, 0)

    def _step(j, _):
      i0 = j * 2
      _gather(i0 + 1, 1)
      _work(i0, 0)

      @pl.when(i0 + 2 < _NITER)
      def _():
        _gather(i0 + 2, 0)
      _work(i0 + 1, 1)
      return 0

    lax.fori_loop(0, _NITER // 2, _step, 0)
    if with_deg:
      pltpu.sync_copy(degp, stage_sh.at[s])
    plsc.subcore_barrier()

    for k in range(_RPT // _ZR):
      pltpu.sync_copy(acc.at[pl.ds(row0 + k * _ZR, _ZR)], zbuf)
      pltpu.sync_copy(zbuf, out_hbm.at[c, pl.ds(row0 + k * _ZR, _ZR)])

    if with_deg:
      pltpu.sync_copy(stage_sh.at[0, pl.ds(row0, _RPT)], degacc)
      for t in range(1, _NS):
        pltpu.sync_copy(stage_sh.at[t, pl.ds(row0, _RPT)], tbuf)
        def _acc(i, _):
          sl = pl.ds(i * 16, 16)
          degacc[sl] = degacc[sl] + tbuf[sl]
          return 0
        lax.fori_loop(0, _RPT // 16, _acc, 0)
      for chunk in range(_RPT // _ZR):
        def _bc(i, _):
          idx = jnp.zeros((16,), jnp.int32) + (chunk * _ZR + i)
          val = plsc.load_gather(degacc, [idx])
          for jj in range(_DH // 16):
            zbuf[i, pl.ds(jj * 16, 16)] = val
          return 0
        lax.fori_loop(0, _ZR, _bc, 0)
        pltpu.sync_copy(
            zbuf, degb_hbm.at[c, pl.ds(row0 + chunk * _ZR, _ZR)])

  return pl.kernel(
      body, out_type=out_type, mesh=mesh, scratch_types=scratch,
      compiler_params=pltpu.CompilerParams(needs_layout_passes=False,
                                           use_tc_tiling_on_sc=False))


_DN = (((1,), (1,)), ((), ()))


def _tc_pre(x_ref, wl_ref, wr_ref, b_ref, y_ref, z_ref):
  xb = x_ref[...]
  y = lax.dot_general(xb, wl_ref[...], _DN, preferred_element_type=jnp.float32)
  y_ref[0] = y[:, :_DH]
  y_ref[1] = y[:, _DH:]
  z_ref[...] = lax.dot_general(xb, wr_ref[...], _DN,
                               preferred_element_type=jnp.float32) + b_ref[...]


def _tc_mid(p_ref, degb_ref, z_ref, gam_ref, bet_ref, wl_ref, wr_ref, b_ref,
            rdeg_ref, y_ref, z1_ref, hpre_v, stats_v):
  g = pl.program_id(0)

  @pl.when(g < _G)
  def _():
    deg_half = degb_ref[0] + degb_ref[1]
    degb = jnp.concatenate([deg_half, deg_half], axis=-1)
    agg = jnp.concatenate([p_ref[0], p_ref[1]], axis=-1)
    rdeg = 1.0 / jnp.maximum(degb, 1.0)
    h = agg * rdeg + z_ref[...]
    hpre_v[pl.ds(g * _BR, _BR), :] = h
    rdeg_ref[...] = rdeg
    st = jnp.concatenate(
        [jnp.sum(h, axis=0)[None], jnp.sum(h * h, axis=0)[None]], axis=0)

    @pl.when(g == 0)
    def _():
      stats_v[...] = st

    @pl.when(g > 0)
    def _():
      stats_v[...] = stats_v[...] + st

  @pl.when(g >= _G)
  def _():
    gp = g - _G
    mean = stats_v[0:1, :] / _N
    var = stats_v[1:2, :] / _N - mean * mean
    scale = lax.rsqrt(var + _EPS) * gam_ref[...]
    h = (hpre_v[pl.ds(gp * _BR, _BR), :] - mean) * scale + bet_ref[...]
    h = jnp.maximum(h, 0.0)
    y = lax.dot_general(h, wl_ref[...], _DN, preferred_element_type=jnp.float32)
    y_ref[0] = y[:, :_DH]
    y_ref[1] = y[:, _DH:]
    z1_ref[...] = lax.dot_general(h, wr_ref[...], _DN,
                                  preferred_element_type=jnp.float32) + b_ref[...]


def _tc_fin(q_ref, rdeg_ref, z_ref, o_ref):
  q = jnp.concatenate([q_ref[0], q_ref[1]], axis=-1)
  o_ref[...] = q * rdeg_ref[...] + z_ref[...]


_row_spec = pl.BlockSpec((_BR, _D), lambda g: (g, 0))
_half_spec = pl.BlockSpec((_NC, _BR, _DH), lambda g: (0, g, 0))
_w_spec = pl.BlockSpec((_D, _D), lambda g: (0, 0))
_v_spec = pl.BlockSpec((1, _D), lambda g: (0, 0))
_st_spec = pl.BlockSpec((2, _D), lambda g: (0, 0))
_f32 = jnp.float32


@jax.jit
def kernel(x, edge_index, W_l0, b_l0, W_r0, gamma0, beta0, W_l1, b_l1, W_r1):
  src = edge_index[0].reshape(_NS, _NITER, _CHUNK)
  dst = edge_index[1].reshape(_NS, _NITER, _CHUNK)

  y0, z0 = pl.pallas_call(
      _tc_pre,
      grid=(_G,),
      in_specs=[_row_spec, _w_spec, _w_spec, _v_spec],
      out_specs=[_half_spec, _row_spec],
      out_shape=[jax.ShapeDtypeStruct((_NC, _N, _DH), _f32),
                 jax.ShapeDtypeStruct((_N, _D), _f32)],
  )(x, W_l0, W_r0, b_l0.reshape(1, _D))

  agg0, degb = _sc_pass(True)(y0, src, dst)

  ga = lambda g: jnp.minimum(g, _G - 1)
  gb = lambda g: jnp.maximum(g - _G, 0)
  rdeg, y1, z1 = pl.pallas_call(
      _tc_mid,
      grid=(2 * _G,),
      in_specs=[pl.BlockSpec((_NC, _BR, _DH), lambda g: (0, ga(g), 0)),
                pl.BlockSpec((_NC, _BR, _DH), lambda g: (0, ga(g), 0)),
                pl.BlockSpec((_BR, _D), lambda g: (ga(g), 0)),
                _v_spec, _v_spec, _w_spec, _w_spec, _v_spec],
      out_specs=[pl.BlockSpec((_BR, _D), lambda g: (ga(g), 0)),
                 pl.BlockSpec((_NC, _BR, _DH), lambda g: (0, gb(g), 0)),
                 pl.BlockSpec((_BR, _D), lambda g: (gb(g), 0))],
      out_shape=[jax.ShapeDtypeStruct((_N, _D), _f32),
                 jax.ShapeDtypeStruct((_NC, _N, _DH), _f32),
                 jax.ShapeDtypeStruct((_N, _D), _f32)],
      scratch_shapes=[pltpu.VMEM((_N, _D), _f32), pltpu.VMEM((2, _D), _f32)],
  )(agg0, degb, z0, gamma0.reshape(1, _D), beta0.reshape(1, _D), W_l1, W_r1,
    b_l1.reshape(1, _D))

  agg1 = _sc_pass(False)(y1, src, dst)[0]

  out = pl.pallas_call(
      _tc_fin,
      grid=(_G,),
      in_specs=[_half_spec, _row_spec, _row_spec],
      out_specs=_row_spec,
      out_shape=jax.ShapeDtypeStruct((_N, _D), _f32),
  )(agg1, rdeg, z1)
  return out


# speedup vs baseline: 1.0072x; 1.0072x over previous
import jax
import jax.numpy as jnp
from jax import lax
from jax.experimental import pallas as pl
from jax.experimental.pallas import tpu as pltpu
from jax.experimental.pallas import tpu_sc as plsc

_N = 10000
_E = 320000
_D = 128
_EPS = 1e-5

_NC = 2
_NS = 16
_CHUNK = 80
_NITER = 250
_EPT = _NITER * _CHUNK
_NP = 10240
_DH = _D // _NC
_RPT = _NP // _NS
_ZR = 128

_BR = 2000
_G = _N // _BR


def _sc_pass(with_deg: bool):
  mesh = plsc.VectorSubcoreMesh(core_axis_name="c", subcore_axis_name="s",
                                num_cores=_NC, num_subcores=_NS)
  out_type = [jax.ShapeDtypeStruct((_NC, _NP, _DH), jnp.float32)]
  scratch = [
      pltpu.VMEM((_NITER, _CHUNK), jnp.int32),
      pltpu.VMEM((_NITER, _CHUNK), jnp.int32),
      [pltpu.VMEM((_CHUNK, _DH), jnp.float32)] * 2,
      pltpu.VMEM((_ZR, _DH), jnp.float32),
      pltpu.VMEM_SHARED((_NP, _DH), jnp.float32),
      [pltpu.SemaphoreType.DMA] * 2,
      [pltpu.SemaphoreType.DMA] * 2,
  ]
  if with_deg:
    out_type.append(jax.ShapeDtypeStruct((_NC, _NP, _DH), jnp.float32))
    scratch += [
        pltpu.VMEM((_NP,), jnp.float32),
        pltpu.VMEM((_RPT,), jnp.float32),
        pltpu.VMEM((_RPT,), jnp.float32),
        pltpu.VMEM_SHARED((_NS, _NP), jnp.float32),
    ]

  def body(y_hbm, src_hbm, dst_hbm, out_hbm, *rest):
    if with_deg:
      (degb_hbm, src_v, dst_v, bufs, zbuf, acc, gsems, ssems,
       degp, degacc, tbuf, stage_sh) = rest
    else:
      src_v, dst_v, bufs, zbuf, acc, gsems, ssems = rest
    c = lax.axis_index("c")
    s = lax.axis_index("s")
    row0 = s * _RPT
    yv = y_hbm.at[c]
    zeros16 = jnp.zeros((16,), jnp.float32)
    ones16 = jnp.ones((16,), jnp.float32)

    def _zb(i, _):
      zbuf[i // (_DH // 16), pl.ds((i % (_DH // 16)) * 16, 16)] = zeros16
      return 0
    lax.fori_loop(0, _ZR * (_DH // 16), _zb, 0)
    for k in range(_RPT // _ZR):
      pltpu.sync_copy(zbuf, acc.at[pl.ds(row0 + k * _ZR, _ZR)])

    if with_deg:
      def _zd(i, _):
        degp[pl.ds(i * 16, 16)] = zeros16
        return 0
      lax.fori_loop(0, _NP // 16, _zd, 0)

    pltpu.sync_copy(src_hbm.at[s], src_v)
    pltpu.sync_copy(dst_hbm.at[s], dst_v)
    plsc.subcore_barrier()

    def _gather(i, b):
      pltpu.async_copy(yv.at[src_v.at[i]], bufs[b], gsems[b])

    def _wait_g(b):
      pltpu.make_async_copy(yv.at[pl.ds(0, _CHUNK)], bufs[b], gsems[b]).wait()

    def _wait_s(b):
      pltpu.make_async_copy(bufs[b], acc.at[pl.ds(0, _CHUNK)], ssems[b]).wait()

    def _work(i, b):
      _wait_g(b)
      pltpu.async_copy(bufs[b], acc.at[dst_v.at[i]], ssems[b], add=True)
      if with_deg:
        @pl.when((i // (_NITER // 2)) == c)
        def _():
          for k in range(_CHUNK // 16):
            d16 = dst_v[i, pl.ds(k * 16, 16)]
            plsc.addupdate_scatter(degp, [d16], ones16)

    _gather(0, 0)

    def _step(j, _):
      i0 = j * 2

      @pl.when(j > 0)
      def _():
        _wait_s(1)
      _gather(i0 + 1, 1)
      _work(i0, 0)

      @pl.when(i0 + 2 < _NITER)
      def _():
        _wait_s(0)
        _gather(i0 + 2, 0)
      _work(i0 + 1, 1)
      return 0

    lax.fori_loop(0, _NITER // 2, _step, 0)
    _wait_s(0)
    _wait_s(1)
    if with_deg:
      pltpu.sync_copy(degp, stage_sh.at[s])
    plsc.subcore_barrier()

    for k in range(_RPT // _ZR):
      pltpu.sync_copy(acc.at[pl.ds(row0 + k * _ZR, _ZR)], zbuf)
      pltpu.sync_copy(zbuf, out_hbm.at[c, pl.ds(row0 + k * _ZR, _ZR)])

    if with_deg:
      pltpu.sync_copy(stage_sh.at[0, pl.ds(row0, _RPT)], degacc)
      for t in range(1, _NS):
        pltpu.sync_copy(stage_sh.at[t, pl.ds(row0, _RPT)], tbuf)
        def _acc(i, _):
          sl = pl.ds(i * 16, 16)
          degacc[sl] = degacc[sl] + tbuf[sl]
          return 0
        lax.fori_loop(0, _RPT // 16, _acc, 0)
      for chunk in range(_RPT // _ZR):
        def _bc(i, _):
          idx = jnp.zeros((16,), jnp.int32) + (chunk * _ZR + i)
          val = plsc.load_gather(degacc, [idx])
          for jj in range(_DH // 16):
            zbuf[i, pl.ds(jj * 16, 16)] = val
          return 0
        lax.fori_loop(0, _ZR, _bc, 0)
        pltpu.sync_copy(
            zbuf, degb_hbm.at[c, pl.ds(row0 + chunk * _ZR, _ZR)])

  return pl.kernel(
      body, out_type=out_type, mesh=mesh, scratch_types=scratch,
      compiler_params=pltpu.CompilerParams(needs_layout_passes=False,
                                           use_tc_tiling_on_sc=False))


_DN = (((1,), (1,)), ((), ()))


def _tc_pre(x_ref, wl_ref, wr_ref, b_ref, y_ref, z_ref):
  xb = x_ref[...]
  y = lax.dot_general(xb, wl_ref[...], _DN, preferred_element_type=jnp.float32)
  y_ref[0] = y[:, :_DH]
  y_ref[1] = y[:, _DH:]
  z_ref[...] = lax.dot_general(xb, wr_ref[...], _DN,
                               preferred_element_type=jnp.float32) + b_ref[...]


def _tc_mid(p_ref, degb_ref, z_ref, gam_ref, bet_ref, wl_ref, wr_ref, b_ref,
            rdeg_ref, y_ref, z1_ref, hpre_v, stats_v):
  g = pl.program_id(0)

  @pl.when(g < _G)
  def _():
    deg_half = degb_ref[0] + degb_ref[1]
    degb = jnp.concatenate([deg_half, deg_half], axis=-1)
    agg = jnp.concatenate([p_ref[0], p_ref[1]], axis=-1)
    rdeg = 1.0 / jnp.maximum(degb, 1.0)
    h = agg * rdeg + z_ref[...]
    hpre_v[pl.ds(g * _BR, _BR), :] = h
    rdeg_ref[...] = rdeg
    st = jnp.concatenate(
        [jnp.sum(h, axis=0)[None], jnp.sum(h * h, axis=0)[None]], axis=0)

    @pl.when(g == 0)
    def _():
      stats_v[...] = st

    @pl.when(g > 0)
    def _():
      stats_v[...] = stats_v[...] + st

  @pl.when(g >= _G)
  def _():
    gp = g - _G
    mean = stats_v[0:1, :] / _N
    var = stats_v[1:2, :] / _N - mean * mean
    scale = lax.rsqrt(var + _EPS) * gam_ref[...]
    h = (hpre_v[pl.ds(gp * _BR, _BR), :] - mean) * scale + bet_ref[...]
    h = jnp.maximum(h, 0.0)
    y = lax.dot_general(h, wl_ref[...], _DN, preferred_element_type=jnp.float32)
    y_ref[0] = y[:, :_DH]
    y_ref[1] = y[:, _DH:]
    z1_ref[...] = lax.dot_general(h, wr_ref[...], _DN,
                                  preferred_element_type=jnp.float32) + b_ref[...]


def _tc_fin(q_ref, rdeg_ref, z_ref, o_ref):
  q = jnp.concatenate([q_ref[0], q_ref[1]], axis=-1)
  o_ref[...] = q * rdeg_ref[...] + z_ref[...]


_row_spec = pl.BlockSpec((_BR, _D), lambda g: (g, 0))
_half_spec = pl.BlockSpec((_NC, _BR, _DH), lambda g: (0, g, 0))
_w_spec = pl.BlockSpec((_D, _D), lambda g: (0, 0))
_v_spec = pl.BlockSpec((1, _D), lambda g: (0, 0))
_st_spec = pl.BlockSpec((2, _D), lambda g: (0, 0))
_f32 = jnp.float32


@jax.jit
def kernel(x, edge_index, W_l0, b_l0, W_r0, gamma0, beta0, W_l1, b_l1, W_r1):
  src = edge_index[0].reshape(_NS, _NITER, _CHUNK)
  dst = edge_index[1].reshape(_NS, _NITER, _CHUNK)

  y0, z0 = pl.pallas_call(
      _tc_pre,
      grid=(_G,),
      in_specs=[_row_spec, _w_spec, _w_spec, _v_spec],
      out_specs=[_half_spec, _row_spec],
      out_shape=[jax.ShapeDtypeStruct((_NC, _N, _DH), _f32),
                 jax.ShapeDtypeStruct((_N, _D), _f32)],
  )(x, W_l0, W_r0, b_l0.reshape(1, _D))

  agg0, degb = _sc_pass(True)(y0, src, dst)

  ga = lambda g: jnp.minimum(g, _G - 1)
  gb = lambda g: jnp.maximum(g - _G, 0)
  rdeg, y1, z1 = pl.pallas_call(
      _tc_mid,
      grid=(2 * _G,),
      in_specs=[pl.BlockSpec((_NC, _BR, _DH), lambda g: (0, ga(g), 0)),
                pl.BlockSpec((_NC, _BR, _DH), lambda g: (0, ga(g), 0)),
                pl.BlockSpec((_BR, _D), lambda g: (ga(g), 0)),
                _v_spec, _v_spec, _w_spec, _w_spec, _v_spec],
      out_specs=[pl.BlockSpec((_BR, _D), lambda g: (ga(g), 0)),
                 pl.BlockSpec((_NC, _BR, _DH), lambda g: (0, gb(g), 0)),
                 pl.BlockSpec((_BR, _D), lambda g: (gb(g), 0))],
      out_shape=[jax.ShapeDtypeStruct((_N, _D), _f32),
                 jax.ShapeDtypeStruct((_NC, _N, _DH), _f32),
                 jax.ShapeDtypeStruct((_N, _D), _f32)],
      scratch_shapes=[pltpu.VMEM((_N, _D), _f32), pltpu.VMEM((2, _D), _f32)],
  )(agg0, degb, z0, gamma0.reshape(1, _D), beta0.reshape(1, _D), W_l1, W_r1,
    b_l1.reshape(1, _D))

  agg1 = _sc_pass(False)(y1, src, dst)[0]

  out = pl.pallas_call(
      _tc_fin,
      grid=(_G,),
      in_specs=[_half_spec, _row_spec, _row_spec],
      out_specs=_row_spec,
      out_shape=jax.ShapeDtypeStruct((_N, _D), _f32),
  )(agg1, rdeg, z1)
  return out
